# baseline (speedup 1.0000x reference)
import functools

import jax
import jax.numpy as jnp
from jax import lax
from jax.experimental import pallas as pl
from jax.experimental.pallas import tpu as pltpu
from jax.experimental.pallas import tpu_sc as plsc

ROWS = 4096
COLS = 16384
W = 512
K = COLS // W
BR = 128
N_SC = 1280
N_TC = ROWS - N_SC
NWORK = 32
RPW = N_SC // NWORK


def _cumsum_block(x_ref, u_ref, o_ref):
    u = u_ref[...]
    carry = jnp.zeros((BR, 1), jnp.float32)
    for q in range(K):
        xq = x_ref[:, q * W:(q + 1) * W].astype(jnp.bfloat16)
        yq = jax.lax.dot_general(
            xq, u,
            dimension_numbers=(((1,), (0,)), ((), ())),
            preferred_element_type=jnp.float32,
        )
        oq = yq + carry
        o_ref[:, q * W:(q + 1) * W] = oq
        carry = oq[:, W - 1:W]


def _tc_part(x_tc):
    i = jax.lax.broadcasted_iota(jnp.int32, (W, W), 0)
    j = jax.lax.broadcasted_iota(jnp.int32, (W, W), 1)
    u_incl = (i <= j).astype(jnp.bfloat16)
    return pl.pallas_call(
        _cumsum_block,
        grid=(N_TC // BR,),
        in_specs=[
            pl.BlockSpec((BR, COLS), lambda i: (i, 0)),
            pl.BlockSpec((W, W), lambda i: (0, 0)),
        ],
        out_specs=pl.BlockSpec((BR, COLS), lambda i: (i, 0)),
        out_shape=jax.ShapeDtypeStruct((N_TC, COLS), jnp.float32),
        compiler_params=pltpu.CompilerParams(
            dimension_semantics=("parallel",),
        ),
    )(x_tc, u_incl)


def _sc_part(x_sc):
    mesh = plsc.VectorSubcoreMesh(core_axis_name="c", subcore_axis_name="s")

    @functools.partial(
        pl.kernel,
        mesh=mesh,
        out_type=jax.ShapeDtypeStruct((N_SC, COLS), jnp.float32),
        scratch_types=[
            pltpu.VMEM((COLS,), jnp.float32),
            pltpu.SemaphoreType.DMA,
        ],
    )
    def sc_copy(x_hbm, o_hbm, buf, sem):
        c = lax.axis_index("c")
        s = lax.axis_index("s")
        wid = s * 2 + c

        def body(i, carry):
            r = wid * RPW + i
            pltpu.sync_copy(x_hbm.at[r], buf)
            pltpu.sync_copy(buf, o_hbm.at[r])
            return carry

        lax.fori_loop(0, RPW, body, 0)

    return sc_copy(x_sc)


@jax.jit
def kernel(x):
    out_tc = _tc_part(x[:N_TC])
    out_sc = _sc_part(x[N_TC:])
    return jnp.concatenate([out_tc, out_sc], axis=0)


# speedup vs baseline: 2.7773x; 2.7773x over previous
import jax
import jax.numpy as jnp
from jax.experimental import pallas as pl
from jax.experimental.pallas import tpu as pltpu

ROWS = 4096
COLS = 16384
W = 256
K = COLS // W
BR = 128


def _cumsum_block(x_ref, u_ref, o_ref):
    u = u_ref[...]
    carry = jnp.zeros((BR, 1), jnp.float32)
    for q in range(K):
        xq = x_ref[:, q * W:(q + 1) * W].astype(jnp.bfloat16)
        yq = jax.lax.dot_general(
            xq, u,
            dimension_numbers=(((1,), (0,)), ((), ())),
            preferred_element_type=jnp.float32,
        )
        oq = yq + carry
        o_ref[:, q * W:(q + 1) * W] = oq
        carry = oq[:, W - 1:W]


@jax.jit
def kernel(x):
    i = jax.lax.broadcasted_iota(jnp.int32, (W, W), 0)
    j = jax.lax.broadcasted_iota(jnp.int32, (W, W), 1)
    u_incl = (i <= j).astype(jnp.bfloat16)
    return pl.pallas_call(
        _cumsum_block,
        grid=(ROWS // BR,),
        in_specs=[
            pl.BlockSpec((BR, COLS), lambda i: (i, 0)),
            pl.BlockSpec((W, W), lambda i: (0, 0)),
        ],
        out_specs=pl.BlockSpec((BR, COLS), lambda i: (i, 0)),
        out_shape=jax.ShapeDtypeStruct((ROWS, COLS), jnp.float32),
        compiler_params=pltpu.CompilerParams(
            dimension_semantics=("parallel",),
        ),
    )(x, u_incl)


# speedup vs baseline: 2.9169x; 1.0503x over previous
import jax
import jax.numpy as jnp
from jax.experimental import pallas as pl
from jax.experimental.pallas import tpu as pltpu

ROWS = 4096
COLS = 16384
NC = 2
CW = COLS // NC
W = 512
K = CW // W
BR = 128


def _cumsum_block(x_ref, u_ref, o_ref, carry_ref):
    @pl.when(pl.program_id(1) == 0)
    def _():
        carry_ref[...] = jnp.zeros((BR, 1), jnp.float32)

    u = u_ref[...]
    carry = carry_ref[...]
    for q in range(K):
        xq = x_ref[:, q * W:(q + 1) * W].astype(jnp.bfloat16)
        yq = jax.lax.dot_general(
            xq, u,
            dimension_numbers=(((1,), (0,)), ((), ())),
            preferred_element_type=jnp.float32,
        )
        oq = yq + carry
        o_ref[:, q * W:(q + 1) * W] = oq
        carry = oq[:, W - 1:W]
    carry_ref[...] = carry


@jax.jit
def kernel(x):
    i = jax.lax.broadcasted_iota(jnp.int32, (W, W), 0)
    j = jax.lax.broadcasted_iota(jnp.int32, (W, W), 1)
    u_incl = (i <= j).astype(jnp.bfloat16)
    return pl.pallas_call(
        _cumsum_block,
        grid=(ROWS // BR, NC),
        in_specs=[
            pl.BlockSpec((BR, CW), lambda i, j: (i, j)),
            pl.BlockSpec((W, W), lambda i, j: (0, 0)),
        ],
        out_specs=pl.BlockSpec((BR, CW), lambda i, j: (i, j)),
        out_shape=jax.ShapeDtypeStruct((ROWS, COLS), jnp.float32),
        scratch_shapes=[pltpu.VMEM((BR, 1), jnp.float32)],
        compiler_params=pltpu.CompilerParams(
            dimension_semantics=("parallel", "arbitrary"),
        ),
    )(x, u_incl)


# speedup vs baseline: 3.0593x; 1.0488x over previous
import jax
import jax.numpy as jnp
from jax.experimental import pallas as pl
from jax.experimental.pallas import tpu as pltpu

ROWS = 4096
COLS = 16384
W = 512
K = COLS // W
BR = 128


def _cumsum_block(x_ref, u_ref, o_ref):
    u = u_ref[...]
    carry = jnp.zeros((BR, 1), jnp.float32)
    for q in range(K):
        xq = x_ref[:, q * W:(q + 1) * W].astype(jnp.bfloat16)
        yq = jax.lax.dot_general(
            xq, u,
            dimension_numbers=(((1,), (0,)), ((), ())),
            preferred_element_type=jnp.float32,
        )
        oq = yq + carry
        o_ref[:, q * W:(q + 1) * W] = oq
        carry = oq[:, W - 1:W]


@jax.jit
def kernel(x):
    i = jax.lax.broadcasted_iota(jnp.int32, (W, W), 0)
    j = jax.lax.broadcasted_iota(jnp.int32, (W, W), 1)
    u_incl = (i <= j).astype(jnp.bfloat16)
    return pl.pallas_call(
        _cumsum_block,
        grid=(ROWS // BR,),
        in_specs=[
            pl.BlockSpec((BR, COLS), lambda i: (i, 0)),
            pl.BlockSpec((W, W), lambda i: (0, 0)),
        ],
        out_specs=pl.BlockSpec((BR, COLS), lambda i: (i, 0)),
        out_shape=jax.ShapeDtypeStruct((ROWS, COLS), jnp.float32),
        compiler_params=pltpu.CompilerParams(
            dimension_semantics=("parallel",),
        ),
    )(x, u_incl)
